# Initial kernel scaffold; baseline (speedup 1.0000x reference)
import jax
import jax.numpy as jnp
from jax.experimental import pallas as pl


def kernel(feats, segment_ids, W, b):
    raise NotImplementedError("write your pallas kernel here")


# speedup vs baseline: 1.4698x; 1.4698x over previous
import functools

import jax
import jax.numpy as jnp
from jax import lax
from jax.experimental import pallas as pl
from jax.experimental.pallas import tpu as pltpu
from jax.experimental.pallas import tpu_sc as plsc

N = 50000
D = 256
G = 512
L = 16
NC = 2
NS = 16
NW = NC * NS
SEG_PER_W = G // NW
C = 80
RG = C // L
DL = D // L


def _make_sc_call():
    mesh = plsc.VectorSubcoreMesh(core_axis_name="c", subcore_axis_name="s")

    @functools.partial(
        pl.kernel,
        mesh=mesh,
        compiler_params=pltpu.CompilerParams(needs_layout_passes=False),
        out_type=[
            jax.ShapeDtypeStruct((G, D), jnp.float32),
            jax.ShapeDtypeStruct((G, D), jnp.float32),
        ],
        scratch_types=[
            pltpu.VMEM((C, D), jnp.float32),
            pltpu.VMEM((C,), jnp.int32),
            pltpu.VMEM((D,), jnp.float32),
            pltpu.VMEM((L,), jnp.float32),
            pltpu.VMEM((NW, L), jnp.int32),
            pltpu.VMEM((SEG_PER_W, D), jnp.float32),
            pltpu.VMEM((SEG_PER_W, D), jnp.float32),
        ],
    )
    def sc_kernel(feats_hbm, ids_hbm, w_hbm, b_hbm, bounds_hbm,
                  out_sum, out_max,
                  rows_v, ids_v, w_v, b_v, bounds_v, sacc, macc):
        wid = lax.axis_index("s") * NC + lax.axis_index("c")
        seg_lo = wid * SEG_PER_W

        pltpu.sync_copy(w_hbm, w_v)
        pltpu.sync_copy(b_hbm, b_v)
        pltpu.sync_copy(bounds_hbm, bounds_v)

        zeros = jnp.zeros((L,), jnp.float32)
        ninf = jnp.full((L,), -jnp.inf, jnp.float32)

        def init_body(g, carry):
            for j in range(DL):
                sacc[g, pl.ds(j * L, L)] = zeros
                macc[g, pl.ds(j * L, L)] = ninf
            return carry

        lax.fori_loop(0, SEG_PER_W, init_body, 0)

        bvec = bounds_v[wid, pl.ds(0, L)]
        cb = bvec[0]
        ce = bvec[1]

        def chunk_body(c, carry):
            r0 = c * C
            pltpu.sync_copy(feats_hbm.at[pl.ds(r0, C)], rows_v)
            pltpu.sync_copy(ids_hbm.at[pl.ds(r0, C)], ids_v)

            def group_body(t, gcarry):
                glo_v = ids_v[pl.ds(t * L, L)] - seg_lo
                for r in range(L):
                    glo = glo_v[r]
                    row = t * L + r

                    @pl.when(jnp.logical_and(glo >= 0, glo < SEG_PER_W))
                    def _():
                        dot = zeros
                        for j in range(DL):
                            dot = dot + (rows_v[row, pl.ds(j * L, L)]
                                         * w_v[pl.ds(j * L, L)])
                        s = jnp.sum(dot)
                        sv = lax.broadcast(s, (L,)) + b_v[...]
                        gate = 1.0 / (1.0 + jnp.exp(-sv))
                        for j in range(DL):
                            rj = rows_v[row, pl.ds(j * L, L)]
                            sacc[glo, pl.ds(j * L, L)] += rj * gate
                            macc[glo, pl.ds(j * L, L)] = jnp.maximum(
                                macc[glo, pl.ds(j * L, L)], rj)

                return gcarry

            lax.fori_loop(0, RG, group_body, 0)
            return carry

        lax.fori_loop(cb, ce, chunk_body, 0)

        pltpu.sync_copy(sacc, out_sum.at[pl.ds(seg_lo, SEG_PER_W)])
        pltpu.sync_copy(macc, out_max.at[pl.ds(seg_lo, SEG_PER_W)])

    return sc_kernel


_SC_CALL = _make_sc_call()


def kernel(feats, segment_ids, W, b):
    w_vec = W.reshape(D).astype(jnp.float32)
    b_vec = jnp.broadcast_to(b.astype(jnp.float32), (L,))
    seg_bounds = jnp.arange(0, G + SEG_PER_W, SEG_PER_W, dtype=jnp.int32)
    starts = jnp.searchsorted(segment_ids, seg_bounds, side="left").astype(jnp.int32)
    chunk_lo = starts[:-1] // C
    chunk_hi = (starts[1:] + C - 1) // C
    bounds = jnp.zeros((NW, L), jnp.int32)
    bounds = bounds.at[:, 0].set(chunk_lo)
    bounds = bounds.at[:, 1].set(chunk_hi)
    out_sum, out_max = _SC_CALL(feats, segment_ids, w_vec, b_vec, bounds)
    return jnp.concatenate([out_sum, out_max], axis=1)


# speedup vs baseline: 2.3463x; 1.5963x over previous
import functools

import jax
import jax.numpy as jnp
from jax import lax
from jax.experimental import pallas as pl
from jax.experimental.pallas import tpu as pltpu
from jax.experimental.pallas import tpu_sc as plsc

N = 50000
D = 256
G = 512
L = 16
NC = 2
NS = 16
NW = NC * NS
SEG_PER_W = G // NW
C = 128
CE = C - 8
DL = D // L

_GDN = lax.GatherDimensionNumbers(
    offset_dims=(), collapsed_slice_dims=(0,), start_index_map=(0,))


def _dyn_bcast(v, i):
    idx = jnp.full((L,), i, jnp.int32)
    return lax.gather(v, idx[:, None], _GDN, (1,),
                      mode=lax.GatherScatterMode.PROMISE_IN_BOUNDS)


def _make_sc_call():
    mesh = plsc.VectorSubcoreMesh(core_axis_name="c", subcore_axis_name="s")

    @functools.partial(
        pl.kernel,
        mesh=mesh,
        compiler_params=pltpu.CompilerParams(needs_layout_passes=False),
        out_type=[
            jax.ShapeDtypeStruct((G, D), jnp.float32),
            jax.ShapeDtypeStruct((G, D), jnp.float32),
        ],
        scratch_types=[
            pltpu.VMEM((C, D), jnp.float32),
            pltpu.VMEM((D,), jnp.float32),
            pltpu.VMEM((L,), jnp.float32),
            pltpu.VMEM((NW, 2 * L), jnp.int32),
            pltpu.VMEM((SEG_PER_W, D), jnp.float32),
            pltpu.VMEM((SEG_PER_W, D), jnp.float32),
        ],
    )
    def sc_kernel(feats_hbm, w_hbm, b_hbm, starts_hbm,
                  out_sum, out_max,
                  rows_v, w_v, b_v, starts_v, sacc, macc):
        wid = lax.axis_index("s") * NC + lax.axis_index("c")
        seg_lo = wid * SEG_PER_W

        pltpu.sync_copy(w_hbm, w_v)
        pltpu.sync_copy(b_hbm, b_v)
        pltpu.sync_copy(starts_hbm, starts_v)

        zeros = jnp.zeros((L,), jnp.float32)
        ninf = jnp.full((L,), -jnp.inf, jnp.float32)
        bvec = b_v[...]

        sb0 = starts_v[wid, pl.ds(0, L)]
        sb1 = starts_v[wid, pl.ds(L, L)]

        def seg_starts(j):
            return sb1[0] if j == L else sb0[j]

        def chunk_body(k, accs, s0, s1):
            start = s0 + k * CE
            base = pl.multiple_of(
                jnp.minimum((start >> 3) << 3, N - C), 8)
            pltpu.sync_copy(feats_hbm.at[pl.ds(base, C)], rows_v)
            lo = start - base
            hi = jnp.minimum(s1, start + CE) - base
            ngr = (hi - lo + L - 1) >> 4

            def group_body(t, accs):
                gb = lo + t * L
                ridx = jnp.minimum(gb + lax.iota(jnp.int32, L), hi - 1)

                def col_body(jg, dot):
                    wv = w_v[pl.ds(jg * L, L)]
                    for jj in range(L):
                        col = jg * L + jj
                        wb = lax.broadcast(wv[jj], (L,))
                        colv = jnp.full((L,), col, jnp.int32)
                        dot = dot + plsc.load_gather(rows_v, [ridx, colv]) * wb
                    return dot

                dot = lax.fori_loop(0, DL, col_body, zeros)
                gate_v = 1.0 / (1.0 + jnp.exp(-(dot + bvec)))

                def row_body(i, accs):
                    sums, maxs = accs
                    g = _dyn_bcast(gate_v, i - gb)
                    new_s, new_m = [], []
                    for j in range(DL):
                        rj = rows_v[i, pl.ds(j * L, L)]
                        new_s.append(sums[j] + rj * g)
                        new_m.append(jnp.maximum(maxs[j], rj))
                    return (tuple(new_s), tuple(new_m))

                return lax.fori_loop(gb, jnp.minimum(gb + L, hi),
                                     row_body, accs)

            return lax.fori_loop(0, ngr, group_body, accs)

        for g_local in range(SEG_PER_W):
            s0 = seg_starts(g_local)
            s1 = seg_starts(g_local + 1)
            nch = (s1 - s0 + CE - 1) // CE
            accs0 = (tuple([zeros] * DL), tuple([ninf] * DL))
            sums, maxs = lax.fori_loop(
                0, nch,
                functools.partial(chunk_body, s0=s0, s1=s1),
                accs0)
            for j in range(DL):
                sacc[g_local, pl.ds(j * L, L)] = sums[j]
                macc[g_local, pl.ds(j * L, L)] = maxs[j]

        pltpu.sync_copy(sacc, out_sum.at[pl.ds(seg_lo, SEG_PER_W)])
        pltpu.sync_copy(macc, out_max.at[pl.ds(seg_lo, SEG_PER_W)])

    return sc_kernel


_SC_CALL = _make_sc_call()


def kernel(feats, segment_ids, W, b):
    w_vec = W.reshape(D).astype(jnp.float32)
    b_vec = jnp.broadcast_to(b.astype(jnp.float32), (L,))
    seg_bounds = jnp.arange(0, G + 1, dtype=jnp.int32)
    starts = jnp.searchsorted(segment_ids, seg_bounds, side="left").astype(jnp.int32)
    win = jnp.arange(NW)[:, None] * SEG_PER_W + jnp.arange(2 * L)[None, :]
    table = starts[jnp.minimum(win, G)]
    out_sum, out_max = _SC_CALL(feats, w_vec, b_vec, table)
    return jnp.concatenate([out_sum, out_max], axis=1)


# speedup vs baseline: 5.0390x; 2.1477x over previous
import functools

import jax
import jax.numpy as jnp
from jax import lax
from jax.experimental import pallas as pl
from jax.experimental.pallas import tpu as pltpu
from jax.experimental.pallas import tpu_sc as plsc

N = 50000
D = 256
G = 512
L = 16
NC = 2
NS = 16
NW = NC * NS
SEG_PER_W = G // NW
C = 128
CE = C - 8
DL = D // L

_GDN = lax.GatherDimensionNumbers(
    offset_dims=(), collapsed_slice_dims=(0,), start_index_map=(0,))


def _dyn_bcast(v, i):
    idx = jnp.full((L,), i, jnp.int32)
    return lax.gather(v, idx[:, None], _GDN, (1,),
                      mode=lax.GatherScatterMode.PROMISE_IN_BOUNDS)


def _make_sc_call():
    mesh = plsc.VectorSubcoreMesh(core_axis_name="c", subcore_axis_name="s")

    @functools.partial(
        pl.kernel,
        mesh=mesh,
        compiler_params=pltpu.CompilerParams(needs_layout_passes=False),
        out_type=[
            jax.ShapeDtypeStruct((G, D), jnp.float32),
            jax.ShapeDtypeStruct((G, D), jnp.float32),
        ],
        scratch_types=[
            pltpu.VMEM((2, C, D), jnp.float32),
            pltpu.VMEM((D,), jnp.float32),
            pltpu.VMEM((L,), jnp.float32),
            pltpu.VMEM((NW, 2 * L), jnp.int32),
            pltpu.VMEM((SEG_PER_W, D), jnp.float32),
            pltpu.VMEM((SEG_PER_W, D), jnp.float32),
            pltpu.SemaphoreType.DMA((2,)),
        ],
    )
    def sc_kernel(feats_hbm, w_hbm, b_hbm, starts_hbm,
                  out_sum, out_max,
                  rows_v, w_v, b_v, starts_v, sacc, macc, sem):
        wid = lax.axis_index("s") * NC + lax.axis_index("c")
        seg_lo = wid * SEG_PER_W

        pltpu.sync_copy(w_hbm, w_v)
        pltpu.sync_copy(b_hbm, b_v)
        pltpu.sync_copy(starts_hbm, starts_v)

        zeros = jnp.zeros((L,), jnp.float32)
        ninf = jnp.full((L,), -jnp.inf, jnp.float32)
        bvec = b_v[...]

        def init_body(g, carry):
            for j in range(DL):
                sacc[g, pl.ds(j * L, L)] = zeros
                macc[g, pl.ds(j * L, L)] = ninf
            return carry

        lax.fori_loop(0, SEG_PER_W, init_body, 0)

        sb0 = starts_v[wid, pl.ds(0, L)]
        sb1 = starts_v[wid, pl.ds(L, L)]
        w0 = sb0[0]
        w1 = sb1[L - 1]
        nq = (w1 - w0 + CE - 1) // CE

        def base_of(q):
            start = w0 + q * CE
            return pl.multiple_of(
                jnp.minimum((start >> 3) << 3, N - C), 8)

        @pl.when(nq > 0)
        def _():
            pltpu.async_copy(feats_hbm.at[pl.ds(base_of(0), C)],
                             rows_v.at[0], sem.at[0])

        def seg_block(gl, lo, hi, par):
            sums = tuple(sacc[gl, pl.ds(j * L, L)] for j in range(DL))
            maxs = tuple(macc[gl, pl.ds(j * L, L)] for j in range(DL))
            ngr = (hi - lo + L - 1) >> 4

            def group_body(t, accs):
                gbase = lo + t * L
                ris = [jnp.minimum(gbase + r, hi - 1) for r in range(L)]

                def colgrp_body(j, dots):
                    wv = w_v[pl.ds(j * L, L)]
                    return tuple(
                        dots[r] + rows_v[par, ris[r], pl.ds(j * L, L)] * wv
                        for r in range(L))

                dots = lax.fori_loop(0, DL, colgrp_body, (zeros,) * L)
                gates = []
                for r in range(L):
                    tot = _dyn_bcast(plsc.cumsum(dots[r]), L - 1)
                    gates.append(1.0 / (1.0 + jnp.exp(-(tot + bvec))))

                sums, maxs = accs
                for r in range(L):
                    valid = (gbase + r) < hi
                    g_r = jnp.where(valid, gates[r], 0.0)
                    new_s, new_m = [], []
                    for j in range(DL):
                        rj = rows_v[par, ris[r], pl.ds(j * L, L)]
                        new_s.append(sums[j] + rj * g_r)
                        new_m.append(jnp.maximum(maxs[j],
                                                 jnp.where(valid, rj, ninf)))
                    sums, maxs = tuple(new_s), tuple(new_m)
                return (sums, maxs)

            sums, maxs = lax.fori_loop(0, ngr, group_body, (sums, maxs))
            for j in range(DL):
                sacc[gl, pl.ds(j * L, L)] = sums[j]
                macc[gl, pl.ds(j * L, L)] = maxs[j]

        def chunk_body(q, carry):
            par = lax.rem(q, 2)
            base = base_of(q)
            pltpu.make_async_copy(feats_hbm.at[pl.ds(base, C)],
                                  rows_v.at[par], sem.at[par]).wait()

            @pl.when(q + 1 < nq)
            def _():
                npar = lax.rem(q + 1, 2)
                pltpu.async_copy(feats_hbm.at[pl.ds(base_of(q + 1), C)],
                                 rows_v.at[npar], sem.at[npar])

            cstart = w0 + q * CE
            cend = jnp.minimum(w1, cstart + CE)
            ga = plsc.all_reduce_population_count(
                sb1 <= jnp.full((L,), cstart, jnp.int32))[0]
            gb = plsc.all_reduce_population_count(
                sb0 < jnp.full((L,), cend, jnp.int32))[0]

            def seg_body(gl, carry2):
                s_g = _dyn_bcast(sb0, gl)[0]
                s_g1 = _dyn_bcast(sb1, gl)[0]
                lo = jnp.maximum(s_g, cstart) - base
                hi = jnp.minimum(s_g1, cend) - base
                seg_block(gl, lo, hi, par)
                return carry2

            lax.fori_loop(ga, gb, seg_body, 0)
            return carry

        lax.fori_loop(0, nq, chunk_body, 0)

        pltpu.sync_copy(sacc, out_sum.at[pl.ds(seg_lo, SEG_PER_W)])
        pltpu.sync_copy(macc, out_max.at[pl.ds(seg_lo, SEG_PER_W)])

    return sc_kernel


_SC_CALL = _make_sc_call()


def kernel(feats, segment_ids, W, b):
    w_vec = W.reshape(D).astype(jnp.float32)
    b_vec = jnp.broadcast_to(b.astype(jnp.float32), (L,))
    seg_bounds = jnp.arange(0, G + 1, dtype=jnp.int32)
    starts = jnp.searchsorted(segment_ids, seg_bounds, side="left").astype(jnp.int32)
    w_base = jnp.arange(NW)[:, None] * SEG_PER_W
    j_off = jnp.arange(L)[None, :]
    table = jnp.concatenate(
        [starts[w_base + j_off], starts[w_base + j_off + 1]], axis=1)
    out_sum, out_max = _SC_CALL(feats, w_vec, b_vec, table)
    return jnp.concatenate([out_sum, out_max], axis=1)
